# baseline (speedup 1.0000x reference)
import jax
import jax.numpy as jnp
from jax.experimental import pallas as pl
from jax.experimental.pallas import tpu as pltpu

NUM_PATCHES = 64
PATCH_AREA = 108
PROJ_DIM = 128
BATCH = 4096

BB = 256
NSTEPS = BATCH // BB
NSPLIT = 4
SUB = BB // NSPLIT


def _enc_kernel(x_hbm, w_ref, pb_ref, o_hbm, x_buf, y_buf, in_sems, out_sems):
    def start_in(i, buf):
        for s in range(NSPLIT):
            pltpu.make_async_copy(
                x_hbm.at[pl.ds(i * BB + s * SUB, SUB)],
                x_buf.at[buf, pl.ds(s * SUB, SUB)],
                in_sems.at[buf, s],
            ).start()

    def wait_in(i, buf):
        for s in range(NSPLIT):
            pltpu.make_async_copy(
                x_hbm.at[pl.ds(i * BB + s * SUB, SUB)],
                x_buf.at[buf, pl.ds(s * SUB, SUB)],
                in_sems.at[buf, s],
            ).wait()

    def start_out(i, buf):
        for s in range(NSPLIT):
            pltpu.make_async_copy(
                y_buf.at[buf, pl.ds(s * SUB, SUB)],
                o_hbm.at[pl.ds(i * BB + s * SUB, SUB)],
                out_sems.at[buf, s],
            ).start()

    def wait_out(i, buf):
        for s in range(NSPLIT):
            pltpu.make_async_copy(
                y_buf.at[buf, pl.ds(s * SUB, SUB)],
                o_hbm.at[pl.ds(i * BB + s * SUB, SUB)],
                out_sems.at[buf, s],
            ).wait()

    w = w_ref[...]
    pb = pb_ref[...]

    start_in(0, 0)
    for i in range(NSTEPS):
        buf = i % 2
        if i + 1 < NSTEPS:
            start_in(i + 1, 1 - buf)
        wait_in(i, buf)
        if i >= 2:
            wait_out(i - 2, buf)
        x = x_buf[buf].reshape(-1, PATCH_AREA)
        y = jax.lax.dot_general(
            x, w,
            dimension_numbers=(((1,), (0,)), ((), ())),
            preferred_element_type=jnp.float32,
        )
        y_buf[buf] = y.reshape(BB, NUM_PATCHES, PROJ_DIM) + pb
        start_out(i, buf)
    wait_out(NSTEPS - 2, NSTEPS % 2)
    wait_out(NSTEPS - 1, (NSTEPS - 1) % 2)


@jax.jit
def kernel(patches, W, b, pos_table):
    pb = (pos_table + b[None, :])[None]
    return pl.pallas_call(
        _enc_kernel,
        in_specs=[
            pl.BlockSpec(memory_space=pltpu.HBM),
            pl.BlockSpec(memory_space=pltpu.VMEM),
            pl.BlockSpec(memory_space=pltpu.VMEM),
        ],
        out_specs=pl.BlockSpec(memory_space=pltpu.HBM),
        out_shape=jax.ShapeDtypeStruct((BATCH, NUM_PATCHES, PROJ_DIM), jnp.float32),
        scratch_shapes=[
            pltpu.VMEM((2, BB, NUM_PATCHES, PATCH_AREA), jnp.float32),
            pltpu.VMEM((2, BB, NUM_PATCHES, PROJ_DIM), jnp.float32),
            pltpu.SemaphoreType.DMA((2, NSPLIT)),
            pltpu.SemaphoreType.DMA((2, NSPLIT)),
        ],
    )(patches, W, pb)


# speedup vs baseline: 1.0862x; 1.0862x over previous
import jax
import jax.numpy as jnp
from jax.experimental import pallas as pl

NUM_PATCHES = 64
PATCH_AREA = 108
PROJ_DIM = 128

BLOCK_ROWS = 16384


def _patch_encoder_kernel(x_ref, w_ref, pb_ref, o_ref):
    y = jax.lax.dot_general(
        x_ref[...], w_ref[...],
        dimension_numbers=(((1,), (0,)), ((), ())),
        preferred_element_type=jnp.float32,
    )
    o_ref[...] = (y.reshape(-1, NUM_PATCHES, PROJ_DIM) + pb_ref[...]).reshape(
        -1, PROJ_DIM
    )


@jax.jit
def kernel(patches, W, b, pos_table):
    batch = patches.shape[0]
    rows = batch * NUM_PATCHES
    x = patches.reshape(rows, PATCH_AREA)
    pb = pos_table + b[None, :]
    grid = (rows // BLOCK_ROWS,)
    out = pl.pallas_call(
        _patch_encoder_kernel,
        grid=grid,
        in_specs=[
            pl.BlockSpec((BLOCK_ROWS, PATCH_AREA), lambda i: (i, 0)),
            pl.BlockSpec((PATCH_AREA, PROJ_DIM), lambda i: (0, 0)),
            pl.BlockSpec((NUM_PATCHES, PROJ_DIM), lambda i: (0, 0)),
        ],
        out_specs=pl.BlockSpec((BLOCK_ROWS, PROJ_DIM), lambda i: (i, 0)),
        out_shape=jax.ShapeDtypeStruct((rows, PROJ_DIM), jnp.float32),
    )(x, W, pb)
    return out.reshape(batch, NUM_PATCHES, PROJ_DIM)
